# baseline (device time: 23745 ns/iter reference)
import os

import jax
import jax.numpy as jnp
from jax import lax
from jax.experimental import pallas as pl
from jax.experimental.pallas import tpu as pltpu

_STAGES = int(os.environ.get("KERNEL_STAGES", "2"))

N_DEV = 16
N_Z = 4
N_P = 4
M_BLK = 64
ROWS_P = 256
N_COL = 1024
HALF = 512
CHUNK = 4
CW = HALF // CHUNK
BF16 = jnp.bfloat16
F32 = jnp.float32


def kernel(x, w_mat):
    def body(x_ref, w_ref, out_ref,
             xp_ref, p_ref, acc_ref,
             rsR, rrR, rsL, rrL,
             ssR, srR, ssL, srL,
             zr_buf, zs_sems, zr_sems, z_bar):
        my = lax.axis_index("i")
        my_z = my // N_P
        my_p = my % N_P
        plane_base = my - my_p
        nbr_r = plane_base + (my_p + 1) % N_P
        nbr_l = plane_base + (my_p - 1) % N_P

        if _STAGES >= 1:
            barrier_sem = pltpu.get_barrier_semaphore()
            for nbr in (nbr_l, nbr_r):
                pl.semaphore_signal(
                    barrier_sem, inc=1,
                    device_id=(nbr,), device_id_type=pl.DeviceIdType.MESH,
                )
            if _STAGES >= 2:
                for j in range(N_Z):
                    @pl.when(j != my_z)
                    def _(j=j):
                        pl.semaphore_signal(
                            z_bar, inc=1,
                            device_id=(N_P * j + my_p,),
                            device_id_type=pl.DeviceIdType.MESH,
                        )
            pl.semaphore_wait(barrier_sem, 2)

        for d in range(N_DEV):
            b = N_P * (d % N_Z) + (d // N_Z)
            xp_ref[M_BLK * d:M_BLK * (d + 1), :] = (
                x_ref[M_BLK * b:M_BLK * (b + 1), :].astype(BF16)
            )
        p_ref[:, :] = jnp.dot(
            xp_ref[:, :], w_ref[:, :].astype(BF16),
            preferred_element_type=F32,
        ).astype(BF16)
        if _STAGES == 0:
            out_ref[:, :] = p_ref[0:M_BLK, :].astype(F32)
            return

        def ring_desc(s, c, q, send_buf, recv_buf, send_sems, recv_sems,
                      nbr, col0):
            k = s * CHUNK + c
            if s == 0:
                src = p_ref.at[pl.ds(q * ROWS_P, ROWS_P),
                               pl.ds(col0 + c * CW, CW)]
            else:
                src = send_buf.at[k]
            return pltpu.make_async_remote_copy(
                src_ref=src, dst_ref=recv_buf.at[k],
                send_sem=send_sems.at[k], recv_sem=recv_sems.at[k],
                device_id=(nbr,), device_id_type=pl.DeviceIdType.MESH,
            )

        def descR(s, c, q=0):
            return ring_desc(s, c, q, rsR, rrR, ssR, srR, nbr_r, HALF)

        def descL(s, c, q=0):
            return ring_desc(s, c, q, rsL, rrL, ssL, srL, nbr_l, 0)

        for s in range(N_P - 1):
            qR = (my_p - s - 1) % N_P
            qL = (my_p + s + 1) % N_P
            rowR = pl.ds(qR * ROWS_P, ROWS_P)
            rowL = pl.ds(qL * ROWS_P, ROWS_P)
            for c in range(CHUNK):
                k = s * CHUNK + c
                colR = slice(HALF + c * CW, HALF + (c + 1) * CW)
                colL = slice(c * CW, (c + 1) * CW)
                if s > 0:
                    descR(s - 1, c, (my_p - s) % N_P).wait_recv()
                    rsR[k, :, :] = (
                        rrR[(s - 1) * CHUNK + c, :, :].astype(F32)
                        + p_ref[rowR, colR].astype(F32)
                    ).astype(BF16)
                    descL(s - 1, c, (my_p + s) % N_P).wait_recv()
                    rsL[k, :, :] = (
                        rrL[(s - 1) * CHUNK + c, :, :].astype(F32)
                        + p_ref[rowL, colL].astype(F32)
                    ).astype(BF16)
                descR(s, c, qR).start()
                descL(s, c, qL).start()

        own_rows = pl.ds(my_p * ROWS_P, ROWS_P)
        s_last = N_P - 2
        for c in range(CHUNK):
            k = s_last * CHUNK + c
            colR = slice(HALF + c * CW, HALF + (c + 1) * CW)
            colL = slice(c * CW, (c + 1) * CW)
            descL(s_last, c).wait_recv()
            acc_ref[:, colL] = (
                p_ref[own_rows, colL].astype(F32) + rrL[k].astype(F32)
            ).astype(BF16)
            descR(s_last, c).wait_recv()
            acc_ref[:, colR] = (
                p_ref[own_rows, colR].astype(F32) + rrR[k].astype(F32)
            ).astype(BF16)

        if _STAGES == 1:
            out_ref[:, :] = acc_ref[pl.ds(my_z * M_BLK, M_BLK), :].astype(F32)
            for s in range(N_P - 1):
                for c in range(CHUNK):
                    descR(s, c, (my_p - s - 1) % N_P).wait_send()
                    descL(s, c, (my_p + s + 1) % N_P).wait_send()
            return

        def z_desc(j, dst_slot, recv_slot):
            return pltpu.make_async_remote_copy(
                src_ref=acc_ref.at[pl.ds(M_BLK * j, M_BLK)],
                dst_ref=zr_buf.at[dst_slot],
                send_sem=zs_sems.at[j], recv_sem=zr_sems.at[recv_slot],
                device_id=(N_P * j + my_p,),
                device_id_type=pl.DeviceIdType.MESH,
            )

        pl.semaphore_wait(z_bar, N_Z - 1)
        for j in range(N_Z):
            @pl.when(j != my_z)
            def _(j=j):
                z_desc(j, my_z, my_z).start()

        out_ref[:, :] = acc_ref[pl.ds(my_z * M_BLK, M_BLK), :].astype(F32)
        for j in range(N_Z):
            @pl.when(j != my_z)
            def _(j=j):
                z_desc(j, j, j).wait_recv()
                out_ref[:, :] = out_ref[:, :] + zr_buf[j, :, :].astype(F32)

        for j in range(N_Z):
            @pl.when(j != my_z)
            def _(j=j):
                z_desc(j, my_z, my_z).wait_send()
        for s in range(N_P - 1):
            for c in range(CHUNK):
                descR(s, c, (my_p - s - 1) % N_P).wait_send()
                descL(s, c, (my_p + s + 1) % N_P).wait_send()

    return pl.pallas_call(
        body,
        out_shape=jax.ShapeDtypeStruct((M_BLK, N_COL), F32),
        in_specs=[
            pl.BlockSpec(memory_space=pltpu.VMEM),
            pl.BlockSpec(memory_space=pltpu.VMEM),
        ],
        out_specs=pl.BlockSpec(memory_space=pltpu.VMEM),
        scratch_shapes=[
            pltpu.VMEM((N_DEV * M_BLK, 64), BF16),
            pltpu.VMEM((N_DEV * M_BLK, N_COL), BF16),
            pltpu.VMEM((ROWS_P, N_COL), BF16),
            pltpu.VMEM(((N_P - 1) * CHUNK, ROWS_P, CW), BF16),
            pltpu.VMEM(((N_P - 1) * CHUNK, ROWS_P, CW), BF16),
            pltpu.VMEM(((N_P - 1) * CHUNK, ROWS_P, CW), BF16),
            pltpu.VMEM(((N_P - 1) * CHUNK, ROWS_P, CW), BF16),
            pltpu.SemaphoreType.DMA(((N_P - 1) * CHUNK,)),
            pltpu.SemaphoreType.DMA(((N_P - 1) * CHUNK,)),
            pltpu.SemaphoreType.DMA(((N_P - 1) * CHUNK,)),
            pltpu.SemaphoreType.DMA(((N_P - 1) * CHUNK,)),
            pltpu.VMEM((N_Z, M_BLK, N_COL), BF16),
            pltpu.SemaphoreType.DMA((N_Z,)),
            pltpu.SemaphoreType.DMA((N_Z,)),
            pltpu.SemaphoreType.REGULAR,
        ],
        compiler_params=pltpu.CompilerParams(
            collective_id=0 if _STAGES >= 1 else None
        ),
    )(x, w_mat)
